# baseline (speedup 1.0000x reference)
import functools

import jax
import jax.numpy as jnp
from jax import lax
from jax.experimental import pallas as pl
from jax.experimental.pallas import tpu as pltpu, tpu_sc as plsc

_NODE_COUNTS = {"question": 50000, "answer": 50000, "comment": 50000, "tag": 10000, "module": 5000}
_RELS = [("tag", "describes", "question"), ("tag", "describes", "answer"), ("tag", "describes", "comment"),
         ("module", "imported_in", "question"), ("module", "imported_in", "answer"),
         ("question", "rev_describes", "tag"), ("answer", "rev_describes", "tag"), ("comment", "rev_describes", "tag"),
         ("question", "rev_imported_in", "module"), ("answer", "rev_imported_in", "module")]
_NT = list(_NODE_COUNTS)
_HID = 128
_NUM_GRAPHS = 64
_MMB = 512
_POOLB = 1000
_E = 60000
_EPAD = 65536
_TROWS = 32
_NPAD = {"question": 50176, "answer": 50176, "comment": 50176, "tag": 10240, "module": 5120}
_NCHUNK = {"question": 4, "answer": 4, "comment": 4, "tag": 1, "module": 1}
_PREG = 648
_TRASH = 640


def _rk(s, r, d):
    return s + "__" + r + "__" + d


def _i16():
    return lax.broadcasted_iota(jnp.int32, (16,), 0)


def _fold_body(w_ref, a_ref, o_ref):
    o_ref[0, 0, :] = jnp.sum(w_ref[0] * a_ref[0, 0][None, :], axis=1)


def _fold_att(w_stack, att_stack):
    k = w_stack.shape[0]
    return pl.pallas_call(
        _fold_body,
        grid=(k,),
        in_specs=[pl.BlockSpec((1, _HID, _HID), lambda i: (i, 0, 0)),
                  pl.BlockSpec((1, 1, _HID), lambda i: (i, 0, 0))],
        out_specs=pl.BlockSpec((1, 1, _HID), lambda i: (i, 0, 0)),
        out_shape=jax.ShapeDtypeStruct((k, 1, _HID), jnp.float32),
    )(w_stack, att_stack[:, None, :])[:, 0, :]


def _mm_multi_body(n_src, n_a, x_ref, *refs):
    w_refs = refs[:n_src]
    ag_ref = refs[n_src]
    o_refs = refs[n_src + 1:n_src + 1 + n_src]
    oa_refs = refs[n_src + 1 + n_src:]
    x = x_ref[...]
    for j in range(n_src):
        o_refs[j][...] = jnp.dot(x, w_refs[j][...], preferred_element_type=jnp.float32)
    at = lax.dot_general(ag_ref[...], x, (((0,), (1,)), ((), ())),
                         preferred_element_type=jnp.float32)
    for j in range(n_a):
        oa_refs[j][...] = at[j]


def _mm_multi(x, w_list, agroup, n_a):
    n, d = x.shape
    n_src = len(w_list)
    return pl.pallas_call(
        functools.partial(_mm_multi_body, n_src, n_a),
        grid=(n // _MMB,),
        in_specs=[pl.BlockSpec((_MMB, d), lambda i: (i, 0))]
                 + [pl.BlockSpec((d, _HID), lambda i: (0, 0))] * n_src
                 + [pl.BlockSpec((d, 8), lambda i: (0, 0))],
        out_specs=[pl.BlockSpec((_MMB, _HID), lambda i: (i, 0))] * n_src
                  + [pl.BlockSpec((_MMB,), lambda i: (i,))] * n_a,
        out_shape=[jax.ShapeDtypeStruct((n, _HID), jnp.float32)] * n_src
                  + [jax.ShapeDtypeStruct((n,), jnp.float32)] * n_a,
    )(x, *w_list, agroup)


def _pool_body(q_ref, b_ref, ps_ref, cnt_ref):
    @pl.when(pl.program_id(0) == 0)
    def _init():
        ps_ref[...] = jnp.zeros_like(ps_ref)
        cnt_ref[...] = jnp.zeros_like(cnt_ref)

    b = b_ref[0, 0]
    onehot = (b[None, :] == lax.broadcasted_iota(jnp.int32, (_NUM_GRAPHS, _POOLB), 0)).astype(jnp.float32)
    ps_ref[...] += jnp.dot(onehot, q_ref[...], preferred_element_type=jnp.float32)
    cnt_ref[...] += jnp.broadcast_to(jnp.sum(onehot, axis=1)[:, None], (_NUM_GRAPHS, _HID))


def _pool(q, batch):
    n = q.shape[0]
    b3 = batch.reshape(n // _POOLB, 1, _POOLB)
    return pl.pallas_call(
        _pool_body,
        grid=(n // _POOLB,),
        in_specs=[pl.BlockSpec((_POOLB, _HID), lambda i: (i, 0)),
                  pl.BlockSpec((1, 1, _POOLB), lambda i: (i, 0, 0))],
        out_specs=[pl.BlockSpec((_NUM_GRAPHS, _HID), lambda i: (0, 0)),
                   pl.BlockSpec((_NUM_GRAPHS, _HID), lambda i: (0, 0))],
        out_shape=[jax.ShapeDtypeStruct((_NUM_GRAPHS, _HID), jnp.float32),
                   jax.ShapeDtypeStruct((_NUM_GRAPHS, _HID), jnp.float32)],
    )(q, b3)


def _head_body(ps_ref, cnt_ref, pe_ref, w_ref, b_ref, o_ref):
    pooled = ps_ref[...] / jnp.maximum(cnt_ref[...], 1.0)
    x = jnp.concatenate([pooled, pe_ref[...]], axis=1)
    logits = jnp.dot(x, w_ref[...], preferred_element_type=jnp.float32) + b_ref[0][None, :]
    m = jnp.max(logits, axis=1, keepdims=True)
    z = jnp.exp(logits - m)
    o_ref[...] = z / jnp.sum(z, axis=1, keepdims=True)


def _head(ps, cnt, post_emb, lin_w, lin_b):
    return pl.pallas_call(
        _head_body,
        out_shape=jax.ShapeDtypeStruct((_NUM_GRAPHS, 2), jnp.float32),
    )(ps, cnt, post_emb, lin_w, lin_b[None, :])


def _sc_group(dst_nt, nrel):
    n_pad = _NPAD[dst_nt]
    half = n_pad // 2
    n_chunks = _NCHUNK[dst_nt]
    ch = half // n_chunks
    share = ch // 16
    piece = 8
    pieces = share // piece
    zshare = n_pad // 16

    mesh = plsc.VectorSubcoreMesh(core_axis_name="c", subcore_axis_name="s")

    scratch = (
        [pltpu.VMEM((_TROWS, 128), jnp.int32)] * nrel
        + [pltpu.VMEM((_TROWS, 128), jnp.int32)] * nrel
        + [pltpu.VMEM((4224,), jnp.float32)] * nrel
        + [
            pltpu.VMEM((128,), jnp.float32),
            pltpu.VMEM((128,), jnp.float32),
            pltpu.VMEM((128,), jnp.float32),
            pltpu.VMEM((192,), jnp.int32),
            pltpu.VMEM((192,), jnp.int32),
            pltpu.VMEM((4224,), jnp.int32),
            pltpu.VMEM((4224,), jnp.int32),
            pltpu.VMEM((4224,), jnp.int32),
            pltpu.VMEM((4224,), jnp.float32),
            pltpu.VMEM((34, 128), jnp.int32),
            pltpu.VMEM((34, 128), jnp.int32),
            pltpu.VMEM((128, 128), jnp.float32),
            pltpu.VMEM((3200,), jnp.float32),
            pltpu.VMEM((16, 128), jnp.float32),
            pltpu.VMEM_SHARED((n_pad,), jnp.float32),
            pltpu.VMEM_SHARED((ch, 128), jnp.float32),
            pltpu.VMEM_SHARED((16 * _PREG,), jnp.int32),
            pltpu.VMEM_SHARED((16 * _PREG,), jnp.float32),
        ]
    )

    def body(*refs):
        rel_refs = [refs[i * 5:(i + 1) * 5] for i in range(nrel)]
        bias_hbm = refs[nrel * 5]
        out_hbm = refs[nrel * 5 + 1]
        sc = refs[nrel * 5 + 2:]
        rsrc = sc[:nrel]
        rdst = sc[nrel:2 * nrel]
        alf = sc[2 * nrel:3 * nrel]
        (avs, avd, dvals, hsA, hsB, posf, packf, ppack_f, palpha_f, psrc_v,
         pdrel_v, rowbuf, zbuf, biasrows, denom, chunkS, pool_p, pool_a) = sc[3 * nrel:]
        c = lax.axis_index("c")
        s = lax.axis_index("s")
        pbase = s * _PREG

        def zrow(i, _):
            zbuf[pl.ds(i * 16, 16)] = jnp.zeros((16,), jnp.float32)
            return 0
        lax.fori_loop(0, 200, zrow, 0)
        for i in range(12):
            hsA[pl.ds(i * 16, 16)] = jnp.zeros((16,), jnp.int32)
            hsB[pl.ds(i * 16, 16)] = jnp.zeros((16,), jnp.int32)
        for k in range(8):
            packf[pl.ds(512 + k * 16, 16)] = jnp.zeros((16,), jnp.int32)
        pltpu.sync_copy(bias_hbm.at[0], biasrows.at[0])

        def bfill(i, _):
            for k in range(8):
                biasrows[i, pl.ds(k * 16, 16)] = biasrows[0, pl.ds(k * 16, 16)]
            return 0
        lax.fori_loop(1, 16, bfill, 0)

        for ri in range(nrel):
            h_hbm, as_hbm, ad_hbm, srcm, dstm = rel_refs[ri]
            pltpu.sync_copy(srcm.at[pl.ds(s * _TROWS, _TROWS), :], rsrc[ri])
            pltpu.sync_copy(dstm.at[pl.ds(s * _TROWS, _TROWS), :], rdst[ri])
            pltpu.sync_copy(zbuf.at[pl.ds(0, zshare)], denom.at[pl.ds(s * zshare, zshare)])
            plsc.subcore_barrier()

            def erow(r, _):
                pltpu.sync_copy(as_hbm.at[rsrc[ri].at[r]], avs)
                pltpu.sync_copy(ad_hbm.at[rdst[ri].at[r]], avd)
                for k in range(8):
                    x = avs[pl.ds(k * 16, 16)] + avd[pl.ds(k * 16, 16)]
                    x = jnp.where(x >= 0.0, x, 0.2 * x)
                    alf[ri][pl.ds(r * 128 + k * 16, 16)] = jnp.exp(x)
                pltpu.sync_copy(alf[ri].at[pl.ds(r * 128, 128)],
                                denom.at[rdst[ri].at[r]], add=True)
                return 0
            lax.fori_loop(0, _TROWS, erow, 0)
            plsc.subcore_barrier()

            def arow(r, _):
                pltpu.sync_copy(denom.at[rdst[ri].at[r]], dvals)
                for k in range(8):
                    e16 = alf[ri][pl.ds(r * 128 + k * 16, 16)]
                    alf[ri][pl.ds(r * 128 + k * 16, 16)] = (
                        e16 / (dvals[pl.ds(k * 16, 16)] + 1e-16))
                return 0
            lax.fori_loop(0, _TROWS, arow, 0)
            for k in range(8):
                alf[ri][pl.ds(4096 + k * 16, 16)] = jnp.zeros((16,), jnp.float32)
            plsc.subcore_barrier()

        def chunk_body(chunk, _):
            lo = c * half + chunk * ch

            def initp(p, _):
                pltpu.sync_copy(biasrows.at[pl.ds(0, piece), :],
                                chunkS.at[pl.ds(s * share + p * piece, piece), :])
                return 0
            lax.fori_loop(0, pieces, initp, 0)
            plsc.subcore_barrier()

            for ri in range(nrel):
                h_hbm = rel_refs[ri][0]

                def hp_body(hp, _):
                    def crow(r, base):
                        rr = hp * 4 + r
                        for k in range(8):
                            d16 = rdst[ri][rr, pl.ds(k * 16, 16)]
                            m = (d16 >= lo) & (d16 < lo + ch)
                            hsA[pl.ds(64 + k * 16, 16)] = jnp.where(m, 1, 0).astype(jnp.int32)
                        bufs = [hsA, hsB]
                        for si, d in enumerate([1, 2, 4, 8, 16, 32, 64]):
                            cur, nxt = bufs[si % 2], bufs[(si + 1) % 2]
                            for k in range(8):
                                nxt[pl.ds(64 + k * 16, 16)] = (cur[pl.ds(64 + k * 16, 16)]
                                                               + cur[pl.ds(64 + k * 16 - d, 16)])
                        cur = bufs[1]
                        total = cur[pl.ds(64 + 112, 16)][15]
                        bb = jnp.broadcast_to(base + pbase - 1, (16,))
                        for k in range(8):
                            d16 = rdst[ri][rr, pl.ds(k * 16, 16)]
                            s16 = rsrc[ri][rr, pl.ds(k * 16, 16)]
                            m = (d16 >= lo) & (d16 < lo + ch)
                            cum = cur[pl.ds(64 + k * 16, 16)]
                            posf[pl.ds(r * 128 + k * 16, 16)] = jnp.where(
                                m, bb + cum, jnp.broadcast_to(pbase + _TRASH, (16,)))
                            packf[pl.ds(r * 128 + k * 16, 16)] = (d16 - lo) * 65536 + s16
                        return base + total
                    cnt = lax.fori_loop(0, 4, crow, jnp.int32(0))

                    gb = jnp.broadcast_to(pbase + cnt, (16,))
                    for k in range(8):
                        posf[pl.ds(512 + k * 16, 16)] = gb + k * 16 + _i16()

                    pltpu.sync_copy(packf.at[pl.ds(0, 640)], pool_p.at[posf.at[pl.ds(0, 640)]])
                    pltpu.sync_copy(alf[ri].at[pl.ds(hp * 512, 512)],
                                    pool_a.at[posf.at[pl.ds(0, 512)]])
                    pltpu.sync_copy(alf[ri].at[pl.ds(4096, 128)],
                                    pool_a.at[posf.at[pl.ds(512, 128)]])
                    pltpu.sync_copy(pool_p.at[pl.ds(pbase, 640)], ppack_f.at[pl.ds(0, 640)])
                    pltpu.sync_copy(pool_a.at[pl.ds(pbase, 640)], palpha_f.at[pl.ds(0, 640)])

                    nb = (cnt + 127) // 128

                    def urow(j, _):
                        for k in range(8):
                            p16 = ppack_f[pl.ds(j * 128 + k * 16, 16)]
                            psrc_v[j, pl.ds(k * 16, 16)] = lax.bitwise_and(p16, 65535)
                            pdrel_v[j, pl.ds(k * 16, 16)] = lax.shift_right_logical(p16, 16)
                        return 0
                    lax.fori_loop(0, nb, urow, 0)

                    def brow(b, _):
                        pltpu.sync_copy(h_hbm.at[psrc_v.at[b]], rowbuf)

                        def rg(g, _):
                            av = palpha_f[pl.ds(b * 128 + g * 16, 16)]
                            for l in range(16):
                                spl = jnp.broadcast_to(av[l], (16,))
                                for k in range(8):
                                    rowbuf[g * 16 + l, pl.ds(k * 16, 16)] = (
                                        rowbuf[g * 16 + l, pl.ds(k * 16, 16)] * spl)
                            return 0
                        lax.fori_loop(0, 8, rg, 0)
                        pltpu.sync_copy(rowbuf, chunkS.at[pdrel_v.at[b]], add=True)
                        return 0
                    lax.fori_loop(0, nb, brow, 0)
                    return 0
                lax.fori_loop(0, 8, hp_body, 0)
            plsc.subcore_barrier()

            def rb(p, _):
                pltpu.sync_copy(chunkS.at[pl.ds(s * share + p * piece, piece), :],
                                rowbuf.at[pl.ds(0, piece), :])

                for rr in range(8):
                    for k in range(8):
                        v = rowbuf[rr, pl.ds(k * 16, 16)]
                        rowbuf[rr, pl.ds(k * 16, 16)] = jnp.maximum(v, 0.0)
                pltpu.sync_copy(rowbuf.at[pl.ds(0, piece), :],
                                out_hbm.at[pl.ds(lo + s * share + p * piece, piece), :])
                return 0
            lax.fori_loop(0, pieces, rb, 0)
            return 0
        lax.fori_loop(0, n_chunks, chunk_body, 0)

    return pl.kernel(
        body,
        out_type=jax.ShapeDtypeStruct((n_pad, _HID), jnp.float32),
        mesh=mesh,
        scratch_types=scratch,
        name="sc_gat_" + dst_nt,
    )


def _pack_groups():
    src_keys = {nt: [] for nt in _NT}
    acols = {nt: [] for nt in _NT}
    for (s, r, d) in _RELS:
        k = _rk(s, r, d)
        src_keys[s].append(k)
        acols[s].append(("src", k))
    for (s, r, d) in _RELS:
        acols[d].append(("dst", _rk(s, r, d)))
    groups = {nt: [] for nt in _NT}
    for (s, r, d) in _RELS:
        groups[d].append((_rk(s, r, d), s))
    return src_keys, acols, groups


_SRC_KEYS, _ACOLS, _GROUPS = _pack_groups()


def _pad_edges(edges):
    out = {}
    for (s, r, d) in _RELS:
        k = _rk(s, r, d)
        e = edges[k]
        sent = _NPAD[d] - 1
        srcp = jnp.concatenate([e[0], jnp.zeros((_EPAD - _E,), jnp.int32)]).reshape(512, 128)
        dstp = jnp.concatenate([e[1], jnp.full((_EPAD - _E,), sent, jnp.int32)]).reshape(512, 128)
        out[k] = (srcp, dstp)
    return out


def kernel(xs, edges, batch, post_emb, params):
    epad = _pad_edges(edges)
    sc_calls = {d_nt: _sc_group(d_nt, len(rels)) for d_nt, rels in _GROUPS.items()}

    h = {nt: jnp.zeros((_NPAD[nt], _HID), jnp.float32).at[:_NODE_COUNTS[nt]].set(xs[nt])
         for nt in _NT}
    for layer in params["layers"]:
        w_list, a_list = [], []
        for nt in _NT:
            for role, k in _ACOLS[nt]:
                w_list.append(layer[k]["W_src" if role == "src" else "W_dst"])
                a_list.append(layer[k]["att_src" if role == "src" else "att_dst"])
        folded = _fold_att(jnp.stack(w_list), jnp.stack(a_list))

        h_of, a_of = {}, {}
        off = 0
        for nt in _NT:
            n_a = len(_ACOLS[nt])
            ag = jnp.zeros((_HID, 8), jnp.float32).at[:, :n_a].set(
                jnp.stack([folded[off + i] for i in range(n_a)], axis=1))
            off += n_a
            outs = _mm_multi(h[nt], [layer[k]["W_src"] for k in _SRC_KEYS[nt]], ag, n_a)
            for j, k in enumerate(_SRC_KEYS[nt]):
                h_of[k] = outs[j]
            for j, role_k in enumerate(_ACOLS[nt]):
                a_of[role_k] = outs[len(_SRC_KEYS[nt]) + j]

        new_h = {}
        for d_nt, rels in _GROUPS.items():
            args = []
            bias_sum = jnp.zeros((_HID,), jnp.float32)
            for (k, s_nt) in rels:
                args += [h_of[k], a_of[("src", k)], a_of[("dst", k)], epad[k][0], epad[k][1]]
                bias_sum = bias_sum + layer[k]["bias"]
            bias8 = jnp.zeros((8, _HID), jnp.float32).at[0].set(bias_sum)
            new_h[d_nt] = sc_calls[d_nt](*args, bias8)
        h = new_h

    ps, cnt = _pool(h["question"][:_NODE_COUNTS["question"]], batch)
    return _head(ps, cnt, post_emb, params["lin_W"], params["lin_b"])


# speedup vs baseline: 1.7615x; 1.7615x over previous
import functools

import jax
import jax.numpy as jnp
from jax import lax
from jax.experimental import pallas as pl
from jax.experimental.pallas import tpu as pltpu, tpu_sc as plsc

_NODE_COUNTS = {"question": 50000, "answer": 50000, "comment": 50000, "tag": 10000, "module": 5000}
_RELS = [("tag", "describes", "question"), ("tag", "describes", "answer"), ("tag", "describes", "comment"),
         ("module", "imported_in", "question"), ("module", "imported_in", "answer"),
         ("question", "rev_describes", "tag"), ("answer", "rev_describes", "tag"), ("comment", "rev_describes", "tag"),
         ("question", "rev_imported_in", "module"), ("answer", "rev_imported_in", "module")]
_NT = list(_NODE_COUNTS)
_HID = 128
_NUM_GRAPHS = 64
_MMB = 512
_POOLB = 1000
_E = 60000
_EPAD = 65536
_TROWS = 32
_NPAD = {"question": 50176, "answer": 50176, "comment": 50176, "tag": 10240, "module": 5120}
_NCHUNK = {"question": 4, "answer": 4, "comment": 4, "tag": 1, "module": 1}
_PREG = 1160
_TRASH = 1152


def _rk(s, r, d):
    return s + "__" + r + "__" + d


def _i16():
    return lax.broadcasted_iota(jnp.int32, (16,), 0)


def _fold_body(w_ref, a_ref, o_ref):
    o_ref[0, 0, :] = jnp.sum(w_ref[0] * a_ref[0, 0][None, :], axis=1)


def _fold_att(w_stack, att_stack):
    k = w_stack.shape[0]
    return pl.pallas_call(
        _fold_body,
        grid=(k,),
        in_specs=[pl.BlockSpec((1, _HID, _HID), lambda i: (i, 0, 0)),
                  pl.BlockSpec((1, 1, _HID), lambda i: (i, 0, 0))],
        out_specs=pl.BlockSpec((1, 1, _HID), lambda i: (i, 0, 0)),
        out_shape=jax.ShapeDtypeStruct((k, 1, _HID), jnp.float32),
    )(w_stack, att_stack[:, None, :])[:, 0, :]


def _mm_multi_body(n_src, n_a, x_ref, *refs):
    w_refs = refs[:n_src]
    ag_ref = refs[n_src]
    o_refs = refs[n_src + 1:n_src + 1 + n_src]
    oa_refs = refs[n_src + 1 + n_src:]
    x = x_ref[...]
    for j in range(n_src):
        o_refs[j][...] = jnp.dot(x, w_refs[j][...], preferred_element_type=jnp.float32)
    at = lax.dot_general(ag_ref[...], x, (((0,), (1,)), ((), ())),
                         preferred_element_type=jnp.float32)
    for j in range(n_a):
        oa_refs[j][...] = at[j]


def _mm_multi(x, w_list, agroup, n_a):
    n, d = x.shape
    n_src = len(w_list)
    return pl.pallas_call(
        functools.partial(_mm_multi_body, n_src, n_a),
        grid=(n // _MMB,),
        in_specs=[pl.BlockSpec((_MMB, d), lambda i: (i, 0))]
                 + [pl.BlockSpec((d, _HID), lambda i: (0, 0))] * n_src
                 + [pl.BlockSpec((d, 8), lambda i: (0, 0))],
        out_specs=[pl.BlockSpec((_MMB, _HID), lambda i: (i, 0))] * n_src
                  + [pl.BlockSpec((_MMB,), lambda i: (i,))] * n_a,
        out_shape=[jax.ShapeDtypeStruct((n, _HID), jnp.float32)] * n_src
                  + [jax.ShapeDtypeStruct((n,), jnp.float32)] * n_a,
    )(x, *w_list, agroup)


def _pool_body(q_ref, b_ref, ps_ref, cnt_ref):
    @pl.when(pl.program_id(0) == 0)
    def _init():
        ps_ref[...] = jnp.zeros_like(ps_ref)
        cnt_ref[...] = jnp.zeros_like(cnt_ref)

    b = b_ref[0, 0]
    onehot = (b[None, :] == lax.broadcasted_iota(jnp.int32, (_NUM_GRAPHS, _POOLB), 0)).astype(jnp.float32)
    ps_ref[...] += jnp.dot(onehot, q_ref[...], preferred_element_type=jnp.float32)
    cnt_ref[...] += jnp.broadcast_to(jnp.sum(onehot, axis=1)[:, None], (_NUM_GRAPHS, _HID))


def _pool(q, batch):
    n = q.shape[0]
    b3 = batch.reshape(n // _POOLB, 1, _POOLB)
    return pl.pallas_call(
        _pool_body,
        grid=(n // _POOLB,),
        in_specs=[pl.BlockSpec((_POOLB, _HID), lambda i: (i, 0)),
                  pl.BlockSpec((1, 1, _POOLB), lambda i: (i, 0, 0))],
        out_specs=[pl.BlockSpec((_NUM_GRAPHS, _HID), lambda i: (0, 0)),
                   pl.BlockSpec((_NUM_GRAPHS, _HID), lambda i: (0, 0))],
        out_shape=[jax.ShapeDtypeStruct((_NUM_GRAPHS, _HID), jnp.float32),
                   jax.ShapeDtypeStruct((_NUM_GRAPHS, _HID), jnp.float32)],
    )(q, b3)


def _head_body(ps_ref, cnt_ref, pe_ref, w_ref, b_ref, o_ref):
    pooled = ps_ref[...] / jnp.maximum(cnt_ref[...], 1.0)
    x = jnp.concatenate([pooled, pe_ref[...]], axis=1)
    logits = jnp.dot(x, w_ref[...], preferred_element_type=jnp.float32) + b_ref[0][None, :]
    m = jnp.max(logits, axis=1, keepdims=True)
    z = jnp.exp(logits - m)
    o_ref[...] = z / jnp.sum(z, axis=1, keepdims=True)


def _head(ps, cnt, post_emb, lin_w, lin_b):
    return pl.pallas_call(
        _head_body,
        out_shape=jax.ShapeDtypeStruct((_NUM_GRAPHS, 2), jnp.float32),
    )(ps, cnt, post_emb, lin_w, lin_b[None, :])


def _sc_group(dst_nt, nrel):
    n_pad = _NPAD[dst_nt]
    half = n_pad // 2
    n_chunks = _NCHUNK[dst_nt]
    ch = half // n_chunks
    share = ch // 16
    piece = 8
    pieces = share // piece
    zshare = n_pad // 16

    mesh = plsc.VectorSubcoreMesh(core_axis_name="c", subcore_axis_name="s")

    scratch = (
        [pltpu.VMEM((_TROWS, 128), jnp.int32)] * nrel
        + [pltpu.VMEM((_TROWS, 128), jnp.int32)] * nrel
        + [pltpu.VMEM((4224,), jnp.float32)] * nrel
        + [
            pltpu.VMEM((128,), jnp.float32),
            pltpu.VMEM((128,), jnp.float32),
            pltpu.VMEM((128,), jnp.float32),
            pltpu.VMEM((192,), jnp.int32),
            pltpu.VMEM((192,), jnp.int32),
            pltpu.VMEM((4224,), jnp.int32),
            pltpu.VMEM((4224,), jnp.int32),
            pltpu.VMEM((4224,), jnp.int32),
            pltpu.VMEM((4224,), jnp.float32),
            pltpu.VMEM((34, 128), jnp.int32),
            pltpu.VMEM((34, 128), jnp.int32),
            pltpu.VMEM((128, 128), jnp.float32),
            pltpu.VMEM((3200,), jnp.float32),
            pltpu.VMEM((16, 128), jnp.float32),
            pltpu.VMEM_SHARED((n_pad,), jnp.float32),
            pltpu.VMEM_SHARED((ch, 128), jnp.float32),
            pltpu.VMEM_SHARED((16 * _PREG,), jnp.int32),
            pltpu.VMEM_SHARED((16 * _PREG,), jnp.float32),
        ]
    )

    def body(*refs):
        rel_refs = [refs[i * 5:(i + 1) * 5] for i in range(nrel)]
        bias_hbm = refs[nrel * 5]
        out_hbm = refs[nrel * 5 + 1]
        sc = refs[nrel * 5 + 2:]
        rsrc = sc[:nrel]
        rdst = sc[nrel:2 * nrel]
        alf = sc[2 * nrel:3 * nrel]
        (avs, avd, dvals, hsA, hsB, posf, packf, ppack_f, palpha_f, psrc_v,
         pdrel_v, rowbuf, zbuf, biasrows, denom, chunkS, pool_p, pool_a) = sc[3 * nrel:]
        c = lax.axis_index("c")
        s = lax.axis_index("s")
        pbase = s * _PREG

        def zrow(i, _):
            zbuf[pl.ds(i * 16, 16)] = jnp.zeros((16,), jnp.float32)
            return 0
        lax.fori_loop(0, 200, zrow, 0)
        for i in range(12):
            hsA[pl.ds(i * 16, 16)] = jnp.zeros((16,), jnp.int32)
            hsB[pl.ds(i * 16, 16)] = jnp.zeros((16,), jnp.int32)
        for k in range(8):
            packf[pl.ds(1024 + k * 16, 16)] = jnp.zeros((16,), jnp.int32)
        pltpu.sync_copy(bias_hbm.at[0], biasrows.at[0])

        def bfill(i, _):
            for k in range(8):
                biasrows[i, pl.ds(k * 16, 16)] = biasrows[0, pl.ds(k * 16, 16)]
            return 0
        lax.fori_loop(1, 16, bfill, 0)

        for ri in range(nrel):
            h_hbm, as_hbm, ad_hbm, srcm, dstm = rel_refs[ri]
            pltpu.sync_copy(srcm.at[pl.ds(s * _TROWS, _TROWS), :], rsrc[ri])
            pltpu.sync_copy(dstm.at[pl.ds(s * _TROWS, _TROWS), :], rdst[ri])
            pltpu.sync_copy(zbuf.at[pl.ds(0, zshare)], denom.at[pl.ds(s * zshare, zshare)])
            plsc.subcore_barrier()

            def erow(r, _):
                pltpu.sync_copy(as_hbm.at[rsrc[ri].at[r]], avs)
                pltpu.sync_copy(ad_hbm.at[rdst[ri].at[r]], avd)
                for k in range(8):
                    x = avs[pl.ds(k * 16, 16)] + avd[pl.ds(k * 16, 16)]
                    x = jnp.where(x >= 0.0, x, 0.2 * x)
                    alf[ri][pl.ds(r * 128 + k * 16, 16)] = jnp.exp(x)
                pltpu.sync_copy(alf[ri].at[pl.ds(r * 128, 128)],
                                denom.at[rdst[ri].at[r]], add=True)
                return 0
            lax.fori_loop(0, _TROWS, erow, 0)
            plsc.subcore_barrier()

            def arow(r, _):
                pltpu.sync_copy(denom.at[rdst[ri].at[r]], dvals)
                for k in range(8):
                    e16 = alf[ri][pl.ds(r * 128 + k * 16, 16)]
                    alf[ri][pl.ds(r * 128 + k * 16, 16)] = (
                        e16 / (dvals[pl.ds(k * 16, 16)] + 1e-16))
                return 0
            lax.fori_loop(0, _TROWS, arow, 0)
            for k in range(8):
                alf[ri][pl.ds(4096 + k * 16, 16)] = jnp.zeros((16,), jnp.float32)
            plsc.subcore_barrier()

        def chunk_body(chunk, _):
            lo = c * half + chunk * ch

            def initp(p, _):
                pltpu.sync_copy(biasrows.at[pl.ds(0, piece), :],
                                chunkS.at[pl.ds(s * share + p * piece, piece), :])
                return 0
            lax.fori_loop(0, pieces, initp, 0)
            plsc.subcore_barrier()

            for ri in range(nrel):
                h_hbm = rel_refs[ri][0]

                def hp_body(hp, _):
                    def crow(r, base):
                        rr = hp * 8 + r
                        for k in range(8):
                            d16 = rdst[ri][rr, pl.ds(k * 16, 16)]
                            m = (d16 >= lo) & (d16 < lo + ch)
                            hsA[pl.ds(64 + k * 16, 16)] = jnp.where(m, 1, 0).astype(jnp.int32)
                        bufs = [hsA, hsB]
                        for si, d in enumerate([1, 2, 4, 8, 16, 32, 64]):
                            cur, nxt = bufs[si % 2], bufs[(si + 1) % 2]
                            for k in range(8):
                                nxt[pl.ds(64 + k * 16, 16)] = (cur[pl.ds(64 + k * 16, 16)]
                                                               + cur[pl.ds(64 + k * 16 - d, 16)])
                        cur = bufs[1]
                        total = cur[pl.ds(64 + 112, 16)][15]
                        bb = jnp.broadcast_to(base + pbase - 1, (16,))
                        for k in range(8):
                            d16 = rdst[ri][rr, pl.ds(k * 16, 16)]
                            s16 = rsrc[ri][rr, pl.ds(k * 16, 16)]
                            m = (d16 >= lo) & (d16 < lo + ch)
                            cum = cur[pl.ds(64 + k * 16, 16)]
                            posf[pl.ds(r * 128 + k * 16, 16)] = jnp.where(
                                m, bb + cum, jnp.broadcast_to(pbase + _TRASH, (16,)))
                            packf[pl.ds(r * 128 + k * 16, 16)] = (d16 - lo) * 65536 + s16
                        return base + total
                    cnt = lax.fori_loop(0, 8, crow, jnp.int32(0))

                    gb = jnp.broadcast_to(pbase + cnt, (16,))
                    for k in range(8):
                        posf[pl.ds(1024 + k * 16, 16)] = gb + k * 16 + _i16()

                    pltpu.sync_copy(packf.at[pl.ds(0, 1152)], pool_p.at[posf.at[pl.ds(0, 1152)]])
                    pltpu.sync_copy(alf[ri].at[pl.ds(hp * 1024, 1024)],
                                    pool_a.at[posf.at[pl.ds(0, 1024)]])
                    pltpu.sync_copy(alf[ri].at[pl.ds(4096, 128)],
                                    pool_a.at[posf.at[pl.ds(1024, 128)]])
                    pltpu.sync_copy(pool_p.at[pl.ds(pbase, 1152)], ppack_f.at[pl.ds(0, 1152)])
                    pltpu.sync_copy(pool_a.at[pl.ds(pbase, 1152)], palpha_f.at[pl.ds(0, 1152)])

                    nb = (cnt + 127) // 128

                    def urow(j, _):
                        for k in range(8):
                            p16 = ppack_f[pl.ds(j * 128 + k * 16, 16)]
                            psrc_v[j, pl.ds(k * 16, 16)] = lax.bitwise_and(p16, 65535)
                            pdrel_v[j, pl.ds(k * 16, 16)] = lax.shift_right_logical(p16, 16)
                        return 0
                    lax.fori_loop(0, nb, urow, 0)

                    def brow(b, _):
                        pltpu.sync_copy(h_hbm.at[psrc_v.at[b]], rowbuf)

                        def rg(g, _):
                            av = palpha_f[pl.ds(b * 128 + g * 16, 16)]
                            for l in range(16):
                                spl = jnp.broadcast_to(av[l], (16,))
                                for k in range(8):
                                    rowbuf[g * 16 + l, pl.ds(k * 16, 16)] = (
                                        rowbuf[g * 16 + l, pl.ds(k * 16, 16)] * spl)
                            return 0
                        lax.fori_loop(0, 8, rg, 0)
                        pltpu.sync_copy(rowbuf, chunkS.at[pdrel_v.at[b]], add=True)
                        return 0
                    lax.fori_loop(0, nb, brow, 0)
                    return 0
                lax.fori_loop(0, 4, hp_body, 0)
            plsc.subcore_barrier()

            def rb(p, _):
                pltpu.sync_copy(chunkS.at[pl.ds(s * share + p * piece, piece), :],
                                rowbuf.at[pl.ds(0, piece), :])

                for rr in range(8):
                    for k in range(8):
                        v = rowbuf[rr, pl.ds(k * 16, 16)]
                        rowbuf[rr, pl.ds(k * 16, 16)] = jnp.maximum(v, 0.0)
                pltpu.sync_copy(rowbuf.at[pl.ds(0, piece), :],
                                out_hbm.at[pl.ds(lo + s * share + p * piece, piece), :])
                return 0
            lax.fori_loop(0, pieces, rb, 0)
            return 0
        lax.fori_loop(0, n_chunks, chunk_body, 0)

    return pl.kernel(
        body,
        out_type=jax.ShapeDtypeStruct((n_pad, _HID), jnp.float32),
        mesh=mesh,
        scratch_types=scratch,
        name="sc_gat_" + dst_nt,
    )


def _pack_groups():
    src_keys = {nt: [] for nt in _NT}
    acols = {nt: [] for nt in _NT}
    for (s, r, d) in _RELS:
        k = _rk(s, r, d)
        src_keys[s].append(k)
        acols[s].append(("src", k))
    for (s, r, d) in _RELS:
        acols[d].append(("dst", _rk(s, r, d)))
    groups = {nt: [] for nt in _NT}
    for (s, r, d) in _RELS:
        groups[d].append((_rk(s, r, d), s))
    return src_keys, acols, groups


_SRC_KEYS, _ACOLS, _GROUPS = _pack_groups()


def _pad_edges(edges):
    out = {}
    for (s, r, d) in _RELS:
        k = _rk(s, r, d)
        e = edges[k]
        sent = _NPAD[d] - 1
        srcp = jnp.concatenate([e[0], jnp.zeros((_EPAD - _E,), jnp.int32)]).reshape(512, 128)
        dstp = jnp.concatenate([e[1], jnp.full((_EPAD - _E,), sent, jnp.int32)]).reshape(512, 128)
        out[k] = (srcp, dstp)
    return out


def kernel(xs, edges, batch, post_emb, params):
    epad = _pad_edges(edges)
    sc_calls = {d_nt: _sc_group(d_nt, len(rels)) for d_nt, rels in _GROUPS.items()}

    h = {nt: jnp.zeros((_NPAD[nt], _HID), jnp.float32).at[:_NODE_COUNTS[nt]].set(xs[nt])
         for nt in _NT}
    for layer in params["layers"]:
        w_list, a_list = [], []
        for nt in _NT:
            for role, k in _ACOLS[nt]:
                w_list.append(layer[k]["W_src" if role == "src" else "W_dst"])
                a_list.append(layer[k]["att_src" if role == "src" else "att_dst"])
        folded = _fold_att(jnp.stack(w_list), jnp.stack(a_list))

        h_of, a_of = {}, {}
        off = 0
        for nt in _NT:
            n_a = len(_ACOLS[nt])
            ag = jnp.zeros((_HID, 8), jnp.float32).at[:, :n_a].set(
                jnp.stack([folded[off + i] for i in range(n_a)], axis=1))
            off += n_a
            outs = _mm_multi(h[nt], [layer[k]["W_src"] for k in _SRC_KEYS[nt]], ag, n_a)
            for j, k in enumerate(_SRC_KEYS[nt]):
                h_of[k] = outs[j]
            for j, role_k in enumerate(_ACOLS[nt]):
                a_of[role_k] = outs[len(_SRC_KEYS[nt]) + j]

        new_h = {}
        for d_nt, rels in _GROUPS.items():
            args = []
            bias_sum = jnp.zeros((_HID,), jnp.float32)
            for (k, s_nt) in rels:
                args += [h_of[k], a_of[("src", k)], a_of[("dst", k)], epad[k][0], epad[k][1]]
                bias_sum = bias_sum + layer[k]["bias"]
            bias8 = jnp.zeros((8, _HID), jnp.float32).at[0].set(bias_sum)
            new_h[d_nt] = sc_calls[d_nt](*args, bias8)
        h = new_h

    ps, cnt = _pool(h["question"][:_NODE_COUNTS["question"]], batch)
    return _head(ps, cnt, post_emb, params["lin_W"], params["lin_b"])
